# Initial kernel scaffold; baseline (speedup 1.0000x reference)
import functools

import jax
import jax.numpy as jnp
from jax import lax
from jax.experimental import pallas as pl
from jax.experimental.pallas import tpu as pltpu
from jax.experimental.pallas import tpu_sc as plsc

_N = 10000
_E = 320000
_D = 128
_NC = 2
_NS = 16
_TILES = _NC * _NS
_EPT = _E // _TILES
_B = 128
_NFULL = _EPT // _B
_RTAIL = _EPT - _NFULL * _B
_RPT = _N // _NS
_ZR = 125
_VPR = _D // 16


def _spmm_body(h_hbm, rows_hbm, cols_hbm, vals_hbm, out_hbm,
               idx_v, ridx_v, vals_v, msg_v,
               idx_t, ridx_t, vals_t, msg_t,
               zbuf_v, acc_sh, sem):
    c = lax.axis_index("c")
    s = lax.axis_index("s")
    tid = c * _NS + s

    def _zrow(i, carry):
        for j in range(_VPR):
            zbuf_v[i, pl.ds(j * 16, 16)] = jnp.zeros((16,), jnp.float32)
        return carry
    lax.fori_loop(0, _ZR, _zrow, 0)
    for k in range(_RPT // _ZR):
        pltpu.sync_copy(zbuf_v, acc_sh.at[pl.ds(s * _RPT + k * _ZR, _ZR)])
    plsc.subcore_barrier()

    ebase = tid * _EPT

    def _do_batch(base, nb, idx, ridx, vals, msg):
        pltpu.sync_copy(cols_hbm.at[pl.ds(base, nb)], idx)
        pltpu.sync_copy(rows_hbm.at[pl.ds(base, nb)], ridx)
        pltpu.sync_copy(vals_hbm.at[pl.ds(base, nb)], vals)
        pltpu.async_copy(h_hbm.at[idx], msg, sem).wait()

        def _scale(k2, carry):
            v = vals[k2]
            for j in range(_VPR):
                sl = pl.ds(j * 16, 16)
                msg[k2, sl] = msg[k2, sl] * v
            return carry
        lax.fori_loop(0, nb, _scale, 0)
        pltpu.sync_copy(msg, acc_sh.at[ridx], add=True)

    def _batch(b, carry):
        _do_batch(ebase + b * _B, _B, idx_v, ridx_v, vals_v, msg_v)
        return carry
    lax.fori_loop(0, _NFULL, _batch, 0)
    _do_batch(ebase + _NFULL * _B, _RTAIL, idx_t, ridx_t, vals_t, msg_t)

    plsc.subcore_barrier()
    for k in range(_RPT // _ZR):
        r0 = s * _RPT + k * _ZR
        pltpu.sync_copy(acc_sh.at[pl.ds(r0, _ZR)], zbuf_v)
        pltpu.sync_copy(zbuf_v, out_hbm.at[c, pl.ds(r0, _ZR)])


def _spmm(h, rows, cols, vals):
    mesh = plsc.VectorSubcoreMesh(
        core_axis_name="c", subcore_axis_name="s",
        num_cores=_NC, num_subcores=_NS)
    return pl.kernel(
        _spmm_body,
        out_type=jax.ShapeDtypeStruct((_NC, _N, _D), jnp.float32),
        mesh=mesh,
        scratch_types=[
            pltpu.VMEM((_B,), jnp.int32),
            pltpu.VMEM((_B,), jnp.int32),
            pltpu.VMEM((_B,), jnp.float32),
            pltpu.VMEM((_B, _D), jnp.float32),
            pltpu.VMEM((_RTAIL,), jnp.int32),
            pltpu.VMEM((_RTAIL,), jnp.int32),
            pltpu.VMEM((_RTAIL,), jnp.float32),
            pltpu.VMEM((_RTAIL, _D), jnp.float32),
            pltpu.VMEM((_ZR, _D), jnp.float32),
            pltpu.VMEM_SHARED((_N, _D), jnp.float32),
            pltpu.SemaphoreType.DMA,
        ],
    )(h, rows, cols, vals)


_BLK = 1000


def _layer_body(xa, xb, w, b, o):
    x = xa[0] + xb[0]
    y = jnp.dot(x, w[...], preferred_element_type=jnp.float32) + b[...]
    o[...] = jnp.where(y >= 0, y, 0.2 * y)


def _layer(parts, w, b):
    return pl.pallas_call(
        _layer_body,
        grid=(_N // _BLK,),
        in_specs=[
            pl.BlockSpec((1, _BLK, _D), lambda i: (0, i, 0)),
            pl.BlockSpec((1, _BLK, _D), lambda i: (1, i, 0)),
            pl.BlockSpec((_D, _D), lambda i: (0, 0)),
            pl.BlockSpec((1, _D), lambda i: (0, 0)),
        ],
        out_specs=pl.BlockSpec((_BLK, _D), lambda i: (i, 0)),
        out_shape=jax.ShapeDtypeStruct((_N, _D), jnp.float32),
    )(parts, parts, w, b.reshape(1, _D))


def _final_body(xa, xb, w1, b1, wmu, bmu, wlv, blv, ini,
                tuned_o, mu_o, lv_o):
    x = xa[0] + xb[0]
    h = jnp.dot(x, w1[...], preferred_element_type=jnp.float32) + b1[...]
    h = jnp.where(h >= 0, h, 0.2 * h)
    mu = jnp.dot(h, wmu[...], preferred_element_type=jnp.float32) + bmu[...]
    lv = jnp.dot(h, wlv[...], preferred_element_type=jnp.float32) + blv[...]
    mu_o[...] = mu
    lv_o[...] = jnp.clip(lv, -20.0, 20.0)
    tuned_o[...] = ini[...] + jnp.where(mu >= 0, mu, 0.25 * mu)


def _final(parts, w1, b1, wmu, bmu, wlv, blv, ini):
    full = pl.BlockSpec((_D, _D), lambda i: (0, 0))
    vec = pl.BlockSpec((1, _D), lambda i: (0, 0))
    blk = pl.BlockSpec((_BLK, _D), lambda i: (i, 0))
    return pl.pallas_call(
        _final_body,
        grid=(_N // _BLK,),
        in_specs=[
            pl.BlockSpec((1, _BLK, _D), lambda i: (0, i, 0)),
            pl.BlockSpec((1, _BLK, _D), lambda i: (1, i, 0)),
            full, vec, full, vec, full, vec, blk,
        ],
        out_specs=(blk, blk, blk),
        out_shape=(
            jax.ShapeDtypeStruct((_N, _D), jnp.float32),
            jax.ShapeDtypeStruct((_N, _D), jnp.float32),
            jax.ShapeDtypeStruct((_N, _D), jnp.float32),
        ),
    )(parts, parts, w1, b1.reshape(1, _D), wmu, bmu.reshape(1, _D),
      wlv, blv.reshape(1, _D), ini)


@jax.jit
def kernel(edge_index, edge_vals, node_feats, ini_embeds,
           W0, b0, W1, b1, Wmu, bmu, Wlv, blv):
    rows = edge_index[0]
    cols = edge_index[1]
    s1 = _spmm(node_feats, rows, cols, edge_vals)
    h1 = _layer(s1, W0, b0)
    s2 = _spmm(h1, rows, cols, edge_vals)
    return _final(s2, W1, b1, Wmu, bmu, Wlv, blv, ini_embeds)


# speedup vs baseline: 5.2045x; 5.2045x over previous
import functools

import jax
import jax.numpy as jnp
from jax import lax
from jax.experimental import pallas as pl
from jax.experimental.pallas import tpu as pltpu
from jax.experimental.pallas import tpu_sc as plsc

_N = 10000
_E = 320000
_D = 128
_NC = 2
_NS = 16
_TILES = _NC * _NS
_EPT = _E // _TILES
_B = 128
_NFULL = _EPT // _B
_RTAIL = _EPT - _NFULL * _B
_NP = 10240
_RPT = _NP // _NS
_ZR = 128
_VPR = _D // 16


def _spmm_body(h_hbm, rows_hbm, cols_hbm, vals_hbm, out_hbm,
               idx_v, ridx_v, vals_v, msg_v,
               idx_t, ridx_t, vals_t, msg_t,
               zbuf_v, acc_sh, sem):
    c = lax.axis_index("c")
    s = lax.axis_index("s")
    tid = c * _NS + s

    def _zrow(i, carry):
        for j in range(_VPR):
            zbuf_v[i, pl.ds(j * 16, 16)] = jnp.zeros((16,), jnp.float32)
        return carry
    lax.fori_loop(0, _ZR, _zrow, 0)
    for k in range(_RPT // _ZR):
        pltpu.sync_copy(zbuf_v, acc_sh.at[pl.ds(s * _RPT + k * _ZR, _ZR)])
    plsc.subcore_barrier()

    ebase = tid * _EPT

    def _do_batch(base, nb, idx, ridx, vals, msg):
        pltpu.sync_copy(cols_hbm.at[pl.ds(base, nb)], idx)
        pltpu.sync_copy(rows_hbm.at[pl.ds(base, nb)], ridx)
        pltpu.sync_copy(vals_hbm.at[pl.ds(base, nb)], vals)
        pltpu.async_copy(h_hbm.at[idx], msg, sem).wait()

        def _scale(g, carry):
            vv = vals[pl.ds(g * 16, 16)]
            for k in range(16):
                v = vv[k]
                r = g * 16 + k
                for j in range(_VPR):
                    sl = pl.ds(j * 16, 16)
                    msg[r, sl] = msg[r, sl] * v
            return carry
        lax.fori_loop(0, nb // 16, _scale, 0)
        pltpu.sync_copy(msg, acc_sh.at[ridx], add=True)

    def _batch(b, carry):
        _do_batch(ebase + b * _B, _B, idx_v, ridx_v, vals_v, msg_v)
        return carry
    lax.fori_loop(0, _NFULL, _batch, 0)
    _do_batch(ebase + _NFULL * _B, _RTAIL, idx_t, ridx_t, vals_t, msg_t)

    plsc.subcore_barrier()
    for k in range(_RPT // _ZR):
        r0 = s * _RPT + k * _ZR
        pltpu.sync_copy(acc_sh.at[pl.ds(r0, _ZR)], zbuf_v)
        pltpu.sync_copy(zbuf_v, out_hbm.at[c, pl.ds(r0, _ZR)])


def _spmm(h, rows, cols, vals):
    mesh = plsc.VectorSubcoreMesh(
        core_axis_name="c", subcore_axis_name="s",
        num_cores=_NC, num_subcores=_NS)
    return pl.kernel(
        _spmm_body,
        out_type=jax.ShapeDtypeStruct((_NC, _NP, _D), jnp.float32),
        mesh=mesh,
        scratch_types=[
            pltpu.VMEM((_B,), jnp.int32),
            pltpu.VMEM((_B,), jnp.int32),
            pltpu.VMEM((_B,), jnp.float32),
            pltpu.VMEM((_B, _D), jnp.float32),
            pltpu.VMEM((_RTAIL,), jnp.int32),
            pltpu.VMEM((_RTAIL,), jnp.int32),
            pltpu.VMEM((_RTAIL,), jnp.float32),
            pltpu.VMEM((_RTAIL, _D), jnp.float32),
            pltpu.VMEM((_ZR, _D), jnp.float32),
            pltpu.VMEM_SHARED((_NP, _D), jnp.float32),
            pltpu.SemaphoreType.DMA,
        ],
    )(h, rows, cols, vals)


_BLK = 1000


def _layer_body(xa, xb, w, b, o):
    x = xa[0] + xb[0]
    y = jnp.dot(x, w[...], preferred_element_type=jnp.float32) + b[...]
    o[...] = jnp.where(y >= 0, y, 0.2 * y)


def _layer(parts, w, b):
    return pl.pallas_call(
        _layer_body,
        grid=(_N // _BLK,),
        in_specs=[
            pl.BlockSpec((1, _BLK, _D), lambda i: (0, i, 0)),
            pl.BlockSpec((1, _BLK, _D), lambda i: (1, i, 0)),
            pl.BlockSpec((_D, _D), lambda i: (0, 0)),
            pl.BlockSpec((1, _D), lambda i: (0, 0)),
        ],
        out_specs=pl.BlockSpec((_BLK, _D), lambda i: (i, 0)),
        out_shape=jax.ShapeDtypeStruct((_N, _D), jnp.float32),
    )(parts, parts, w, b.reshape(1, _D))


def _final_body(xa, xb, w1, b1, wmu, bmu, wlv, blv, ini,
                tuned_o, mu_o, lv_o):
    x = xa[0] + xb[0]
    h = jnp.dot(x, w1[...], preferred_element_type=jnp.float32) + b1[...]
    h = jnp.where(h >= 0, h, 0.2 * h)
    mu = jnp.dot(h, wmu[...], preferred_element_type=jnp.float32) + bmu[...]
    lv = jnp.dot(h, wlv[...], preferred_element_type=jnp.float32) + blv[...]
    mu_o[...] = mu
    lv_o[...] = jnp.clip(lv, -20.0, 20.0)
    tuned_o[...] = ini[...] + jnp.where(mu >= 0, mu, 0.25 * mu)


def _final(parts, w1, b1, wmu, bmu, wlv, blv, ini):
    full = pl.BlockSpec((_D, _D), lambda i: (0, 0))
    vec = pl.BlockSpec((1, _D), lambda i: (0, 0))
    blk = pl.BlockSpec((_BLK, _D), lambda i: (i, 0))
    return pl.pallas_call(
        _final_body,
        grid=(_N // _BLK,),
        in_specs=[
            pl.BlockSpec((1, _BLK, _D), lambda i: (0, i, 0)),
            pl.BlockSpec((1, _BLK, _D), lambda i: (1, i, 0)),
            full, vec, full, vec, full, vec, blk,
        ],
        out_specs=(blk, blk, blk),
        out_shape=(
            jax.ShapeDtypeStruct((_N, _D), jnp.float32),
            jax.ShapeDtypeStruct((_N, _D), jnp.float32),
            jax.ShapeDtypeStruct((_N, _D), jnp.float32),
        ),
    )(parts, parts, w1, b1.reshape(1, _D), wmu, bmu.reshape(1, _D),
      wlv, blv.reshape(1, _D), ini)


@jax.jit
def kernel(edge_index, edge_vals, node_feats, ini_embeds,
           W0, b0, W1, b1, Wmu, bmu, Wlv, blv):
    rows = edge_index[0]
    cols = edge_index[1]
    s1 = _spmm(node_feats, rows, cols, edge_vals)
    h1 = _layer(s1, W0, b0)
    s2 = _spmm(h1, rows, cols, edge_vals)
    return _final(s2, W1, b1, Wmu, bmu, Wlv, blv, ini_embeds)
